# baseline (speedup 1.0000x reference)
import functools
import math

import jax
import jax.numpy as jnp
from jax import lax
from jax.experimental import pallas as pl
from jax.experimental.pallas import tpu as pltpu
from jax.experimental.pallas import tpu_sc as plsc

D_MODEL = 64
SCALE = math.sqrt(D_MODEL)
LANES = 16
NC, NS = 2, 16
NW = NC * NS
B_ROWS = 4096
SEQ = 200
BT = B_ROWS // NW
DB = 8
RING = 3


def _emb_body(x_hbm, table_hbm, out_hbm, idx_v, b0, b1, b2, t0, t1, t2,
              g0, g1, g2, s0, s1, s2):
    bufs = (b0, b1, b2)
    tbufs = (t0, t1, t2)
    gsems = (g0, g1, g2)
    ssems = (s0, s1, s2)
    wid = lax.axis_index("s") * NC + lax.axis_index("c")
    pltpu.sync_copy(x_hbm.at[:, pl.ds(wid * BT, BT)], idx_v)

    def fire_gather(s, slot):
        pltpu.async_copy(table_hbm.at[idx_v.at[s]], bufs[slot], gsems[slot])

    def wait_gather(slot):
        pltpu.make_async_copy(
            table_hbm.at[idx_v.at[0]], bufs[slot], gsems[slot]
        ).wait()

    def fire_store(s, slot):
        for db in range(DB):
            pltpu.async_copy(
                tbufs[slot].at[db, :, pl.ds(0, BT)],
                out_hbm.at[s, db, wid],
                ssems[slot],
            )

    def wait_store(slot):
        for db in range(DB):
            pltpu.make_async_copy(
                tbufs[slot].at[db, :, pl.ds(0, BT)],
                out_hbm.at[0, db, wid],
                ssems[slot],
            ).wait()

    def transpose_scale(slot, tslot):
        buf = bufs[slot]
        tbuf = tbufs[tslot]
        didx = [
            k * LANES + lax.iota(jnp.int32, LANES)
            for k in range(D_MODEL // LANES)
        ]
        dbv = [d >> 3 for d in didx]
        d8v = [d & 7 for d in didx]

        @plsc.parallel_loop(0, BT, unroll=16)
        def _(r):
            col = jnp.full((LANES,), r, jnp.int32)
            for k in range(D_MODEL // LANES):
                vals = buf[r, pl.ds(k * LANES, LANES)] * SCALE
                plsc.store_scatter(tbuf, [dbv[k], d8v[k], col], vals)

    fire_gather(0, 0)
    fire_gather(1, 1)

    def outer(t, carry):
        for p in range(RING):
            g = t * RING + p
            nslot = (p + 2) % RING

            @pl.when(g < SEQ)
            def _():
                wait_gather(p)

                @pl.when(g >= RING)
                def _():
                    wait_store(p)

                transpose_scale(p, p)

                @pl.when(g + 2 < SEQ)
                def _():
                    fire_gather(g + 2, nslot)

                fire_store(g, p)

        return carry

    lax.fori_loop(0, (SEQ + RING - 1) // RING, outer, 0)
    wait_store((SEQ - 1) % RING)
    wait_store((SEQ - 2) % RING)
    wait_store((SEQ - 3) % RING)


@functools.cache
def _build():
    mesh = plsc.VectorSubcoreMesh(
        core_axis_name="c", subcore_axis_name="s", num_cores=NC, num_subcores=NS
    )
    return functools.partial(
        pl.kernel,
        out_type=jax.ShapeDtypeStruct(
            (SEQ, DB, NW, D_MODEL // DB, BT), jnp.float32
        ),
        mesh=mesh,
        scratch_types=[
            pltpu.VMEM((SEQ, BT), jnp.int32),
            pltpu.VMEM((BT, D_MODEL), jnp.float32),
            pltpu.VMEM((BT, D_MODEL), jnp.float32),
            pltpu.VMEM((BT, D_MODEL), jnp.float32),
            pltpu.VMEM((DB, D_MODEL // DB, BT + 1), jnp.float32),
            pltpu.VMEM((DB, D_MODEL // DB, BT + 1), jnp.float32),
            pltpu.VMEM((DB, D_MODEL // DB, BT + 1), jnp.float32),
            pltpu.SemaphoreType.DMA,
            pltpu.SemaphoreType.DMA,
            pltpu.SemaphoreType.DMA,
            pltpu.SemaphoreType.DMA,
            pltpu.SemaphoreType.DMA,
            pltpu.SemaphoreType.DMA,
        ],
        compiler_params=pltpu.CompilerParams(
            use_tc_tiling_on_sc=False, needs_layout_passes=False
        ),
    )(_emb_body)


def kernel(x, lut_weight):
    out5 = _build()(x.T.astype(jnp.int32), lut_weight)
    return out5.transpose((2, 4, 0, 1, 3)).reshape(B_ROWS, SEQ, D_MODEL)


# speedup vs baseline: 1.0346x; 1.0346x over previous
import functools
import math

import jax
import jax.numpy as jnp
from jax import lax
from jax.experimental import pallas as pl
from jax.experimental.pallas import tpu as pltpu
from jax.experimental.pallas import tpu_sc as plsc

D_MODEL = 64
SCALE = math.sqrt(D_MODEL)
LANES = 16
NC, NS = 2, 16
NW = NC * NS
B_ROWS = 4096
SEQ = 200
BT = B_ROWS // NW
DB = 8
RING = 3


def _emb_body(x_hbm, table_hbm, out_hbm, idx_v, b0, b1, b2, t0, t1, t2,
              g0, g1, g2, s0, s1, s2):
    bufs = (b0, b1, b2)
    tbufs = (t0, t1, t2)
    gsems = (g0, g1, g2)
    ssems = (s0, s1, s2)
    wid = lax.axis_index("s") * NC + lax.axis_index("c")
    pltpu.sync_copy(x_hbm.at[:, pl.ds(wid * BT, BT)], idx_v)

    def fire_gather(s, slot):
        pltpu.async_copy(table_hbm.at[idx_v.at[s]], bufs[slot], gsems[slot])

    def wait_gather(slot):
        pltpu.make_async_copy(
            table_hbm.at[idx_v.at[0]], bufs[slot], gsems[slot]
        ).wait()

    def fire_store(s, slot):
        for db in range(DB):
            pltpu.async_copy(
                tbufs[slot].at[db, :, pl.ds(0, BT)],
                out_hbm.at[s, db, wid],
                ssems[slot],
            )

    def wait_store(slot):
        for db in range(DB):
            pltpu.make_async_copy(
                tbufs[slot].at[db, :, pl.ds(0, BT)],
                out_hbm.at[0, db, wid],
                ssems[slot],
            ).wait()

    def transpose_scale(slot, tslot):
        buf = bufs[slot]
        tbuf = tbufs[tslot]
        didx = [
            k * LANES + lax.iota(jnp.int32, LANES)
            for k in range(D_MODEL // LANES)
        ]
        dbv = [d >> 3 for d in didx]
        d8v = [d & 7 for d in didx]

        @plsc.parallel_loop(0, BT, unroll=8)
        def _(r):
            col = jnp.full((LANES,), r, jnp.int32)
            for k in range(D_MODEL // LANES):
                vals = buf[r, pl.ds(k * LANES, LANES)] * SCALE
                plsc.store_scatter(tbuf, [dbv[k], d8v[k], col], vals)

    fire_gather(0, 0)
    fire_gather(1, 1)

    def outer(t, carry):
        for p in range(RING):
            g = t * RING + p
            nslot = (p + 2) % RING

            @pl.when(g < SEQ)
            def _():
                @pl.when(g + 2 < SEQ)
                def _():
                    fire_gather(g + 2, nslot)

                wait_gather(p)

                @pl.when(g >= RING)
                def _():
                    wait_store(p)

                transpose_scale(p, p)
                fire_store(g, p)

        return carry

    lax.fori_loop(0, (SEQ + RING - 1) // RING, outer, 0)
    wait_store((SEQ - 1) % RING)
    wait_store((SEQ - 2) % RING)
    wait_store((SEQ - 3) % RING)


@functools.cache
def _build():
    mesh = plsc.VectorSubcoreMesh(
        core_axis_name="c", subcore_axis_name="s", num_cores=NC, num_subcores=NS
    )
    return functools.partial(
        pl.kernel,
        out_type=jax.ShapeDtypeStruct(
            (SEQ, DB, NW, D_MODEL // DB, BT), jnp.float32
        ),
        mesh=mesh,
        scratch_types=[
            pltpu.VMEM((SEQ, BT), jnp.int32),
            pltpu.VMEM((BT, D_MODEL), jnp.float32),
            pltpu.VMEM((BT, D_MODEL), jnp.float32),
            pltpu.VMEM((BT, D_MODEL), jnp.float32),
            pltpu.VMEM((DB, D_MODEL // DB, BT + 1), jnp.float32),
            pltpu.VMEM((DB, D_MODEL // DB, BT + 1), jnp.float32),
            pltpu.VMEM((DB, D_MODEL // DB, BT + 1), jnp.float32),
            pltpu.SemaphoreType.DMA,
            pltpu.SemaphoreType.DMA,
            pltpu.SemaphoreType.DMA,
            pltpu.SemaphoreType.DMA,
            pltpu.SemaphoreType.DMA,
            pltpu.SemaphoreType.DMA,
        ],
        compiler_params=pltpu.CompilerParams(
            use_tc_tiling_on_sc=False, needs_layout_passes=False
        ),
    )(_emb_body)


def kernel(x, lut_weight):
    out5 = _build()(x.T.astype(jnp.int32), lut_weight)
    return out5.transpose((2, 4, 0, 1, 3)).reshape(B_ROWS, SEQ, D_MODEL)


# speedup vs baseline: 1.0462x; 1.0112x over previous
import functools
import math

import jax
import jax.numpy as jnp
from jax import lax
from jax.experimental import pallas as pl
from jax.experimental.pallas import tpu as pltpu
from jax.experimental.pallas import tpu_sc as plsc

D_MODEL = 64
SCALE = math.sqrt(D_MODEL)
LANES = 16
NC, NS = 2, 16
NW = NC * NS
B_ROWS = 4096
SEQ = 200
BT = B_ROWS // NW
DB = 8
RING = 4


def _emb_body(x_hbm, table_hbm, out_hbm, idx_v, b0, b1, b2, b3, t0, t1, t2,
              t3, g0, g1, g2, g3, s0, s1, s2, s3):
    bufs = (b0, b1, b2, b3)
    tbufs = (t0, t1, t2, t3)
    gsems = (g0, g1, g2, g3)
    ssems = (s0, s1, s2, s3)
    wid = lax.axis_index("s") * NC + lax.axis_index("c")
    pltpu.sync_copy(x_hbm.at[:, pl.ds(wid * BT, BT)], idx_v)

    def fire_gather(s, slot):
        pltpu.async_copy(table_hbm.at[idx_v.at[s]], bufs[slot], gsems[slot])

    def wait_gather(slot):
        pltpu.make_async_copy(
            table_hbm.at[idx_v.at[0]], bufs[slot], gsems[slot]
        ).wait()

    def fire_store(s, slot):
        for db in range(DB):
            pltpu.async_copy(
                tbufs[slot].at[db, :, pl.ds(0, BT)],
                out_hbm.at[s, db, wid],
                ssems[slot],
            )

    def wait_store(slot):
        for db in range(DB):
            pltpu.make_async_copy(
                tbufs[slot].at[db, :, pl.ds(0, BT)],
                out_hbm.at[0, db, wid],
                ssems[slot],
            ).wait()

    def transpose_scale(slot, tslot):
        buf = bufs[slot]
        tbuf = tbufs[tslot]
        didx = [
            k * LANES + lax.iota(jnp.int32, LANES)
            for k in range(D_MODEL // LANES)
        ]
        dbv = [d >> 3 for d in didx]
        d8v = [d & 7 for d in didx]

        @plsc.parallel_loop(0, BT, unroll=8)
        def _(r):
            col = jnp.full((LANES,), r, jnp.int32)
            for k in range(D_MODEL // LANES):
                vals = buf[r, pl.ds(k * LANES, LANES)] * SCALE
                plsc.store_scatter(tbuf, [dbv[k], d8v[k], col], vals)

    fire_gather(0, 0)
    fire_gather(1, 1)
    fire_gather(2, 2)

    def outer(t, carry):
        for p in range(RING):
            g = t * RING + p
            nslot = (p + 3) % RING

            @pl.when(g < SEQ)
            def _():
                @pl.when(g + 3 < SEQ)
                def _():
                    fire_gather(g + 3, nslot)

                wait_gather(p)

                @pl.when(g >= RING)
                def _():
                    wait_store(p)

                transpose_scale(p, p)
                fire_store(g, p)

        return carry

    lax.fori_loop(0, (SEQ + RING - 1) // RING, outer, 0)
    wait_store((SEQ - 1) % RING)
    wait_store((SEQ - 2) % RING)
    wait_store((SEQ - 3) % RING)
    wait_store((SEQ - 4) % RING)


@functools.cache
def _build():
    mesh = plsc.VectorSubcoreMesh(
        core_axis_name="c", subcore_axis_name="s", num_cores=NC, num_subcores=NS
    )
    return functools.partial(
        pl.kernel,
        out_type=jax.ShapeDtypeStruct(
            (SEQ, DB, NW, D_MODEL // DB, BT), jnp.float32
        ),
        mesh=mesh,
        scratch_types=[
            pltpu.VMEM((SEQ, BT), jnp.int32),
            pltpu.VMEM((BT, D_MODEL), jnp.float32),
            pltpu.VMEM((BT, D_MODEL), jnp.float32),
            pltpu.VMEM((BT, D_MODEL), jnp.float32),
            pltpu.VMEM((BT, D_MODEL), jnp.float32),
            pltpu.VMEM((DB, D_MODEL // DB, BT + 1), jnp.float32),
            pltpu.VMEM((DB, D_MODEL // DB, BT + 1), jnp.float32),
            pltpu.VMEM((DB, D_MODEL // DB, BT + 1), jnp.float32),
            pltpu.VMEM((DB, D_MODEL // DB, BT + 1), jnp.float32),
            pltpu.SemaphoreType.DMA,
            pltpu.SemaphoreType.DMA,
            pltpu.SemaphoreType.DMA,
            pltpu.SemaphoreType.DMA,
            pltpu.SemaphoreType.DMA,
            pltpu.SemaphoreType.DMA,
            pltpu.SemaphoreType.DMA,
            pltpu.SemaphoreType.DMA,
        ],
        compiler_params=pltpu.CompilerParams(
            use_tc_tiling_on_sc=False, needs_layout_passes=False
        ),
    )(_emb_body)


def kernel(x, lut_weight):
    out5 = _build()(x.T.astype(jnp.int32), lut_weight)
    return out5.transpose((2, 4, 0, 1, 3)).reshape(B_ROWS, SEQ, D_MODEL)
